# Initial kernel scaffold; baseline (speedup 1.0000x reference)
import jax
import jax.numpy as jnp
from jax.experimental import pallas as pl


def kernel(x):
    raise NotImplementedError("write your pallas kernel here")


# speedup vs baseline: 2.7408x; 2.7408x over previous
import functools

import jax
import jax.numpy as jnp
import numpy as np
from jax import lax
from jax.experimental import pallas as pl
from jax.experimental.pallas import tpu as pltpu
from jax.experimental.pallas import tpu_sc as plsc

NUM_EXPERTS = 16
ROWS = 16384
COLS = 2048
LANES = 16
VECS_PER_ROW = COLS // LANES

_PI = np.float32(np.pi)
_TWO_PI = np.float32(2.0) * _PI
_HALF_PI = _PI / np.float32(2.0)
_SIN_PI = np.float32(np.sin(np.float32(np.pi)))
_INV_C = np.float32(1.0) / np.float32(COLS)

NW = 32
ROWS_PER_W = ROWS // NW
CH = 8
NCH = ROWS_PER_W // CH


def _row_negcount(buf, rr):
    def body(j, acc):
        v = buf[rr, pl.ds(j * LANES, LANES)]
        return acc + lax.shift_right_logical(v, 31)

    acc0 = jnp.zeros((LANES,), jnp.int32)
    return lax.fori_loop(0, VECS_PER_ROW, body, acc0, unroll=8)


def _phase_router_kernel(x_hbm, probs_hbm, idx_hbm, buf, accs, probs_s, idx_s,
                         sem0, sem1):
    ci = lax.axis_index("c")
    si = lax.axis_index("s")
    wid = si * 2 + ci
    base_row = wid * ROWS_PER_W

    lane = lax.broadcasted_iota(jnp.int32, (LANES,), 0)
    ones = jnp.ones((LANES,), jnp.float32)

    def dma_in(c, b, sem):
        return pltpu.make_async_copy(
            x_hbm.at[pl.ds(base_row + c * CH, CH)], buf.at[b], sem)

    dma_in(0, 0, sem0).start()
    dma_in(1, 1, sem1).start()

    def process_chunk(c, b, sem):
        dma_in(c, b, sem).wait()
        for rr in range(CH):
            accs[rr, :] = _row_negcount(buf.at[b], rr)
        counts = jnp.zeros((LANES,), jnp.int32)
        for j in range(LANES):
            col = jnp.full((LANES,), j, jnp.int32)
            counts = counts + plsc.load_gather(accs, [lane, col])

        nf = counts.astype(jnp.float32)
        d = jnp.float32(COLS) - jnp.float32(2.0) * nf
        mc = d * _INV_C
        ms = (nf * _SIN_PI) * _INV_C
        ratio = ms / mc
        tp = jnp.where(d > 0, ratio,
                       jnp.where(d < 0, ratio - _PI, -_HALF_PI))
        norm = (tp + _PI) / _TWO_PI
        idx = (norm * jnp.float32(NUM_EXPERTS)).astype(jnp.int32)
        idx = jnp.clip(idx, 0, NUM_EXPERTS - 1)

        row0 = c * CH
        live = lane < CH
        plsc.store_scatter(idx_s, [row0 + lane], idx, mask=live)
        for rr in range(CH):
            probs_s[row0 + rr, :] = jnp.zeros((LANES,), jnp.float32)
        plsc.store_scatter(probs_s, [row0 + lane, idx], ones, mask=live)

    def outer(t, _):
        c0 = 2 * t
        process_chunk(c0, 0, sem0)

        @pl.when(c0 + 2 < NCH)
        def _():
            dma_in(c0 + 2, 0, sem0).start()

        process_chunk(c0 + 1, 1, sem1)

        @pl.when(c0 + 3 < NCH)
        def _():
            dma_in(c0 + 3, 1, sem1).start()

        return 0

    lax.fori_loop(0, NCH // 2, outer, 0)

    pltpu.sync_copy(probs_s, probs_hbm.at[pl.ds(base_row, ROWS_PER_W)])
    pltpu.sync_copy(idx_s, idx_hbm.at[pl.ds(base_row, ROWS_PER_W)])


@jax.jit
def kernel(x):
    xi = lax.bitcast_convert_type(x, jnp.int32)
    mesh = plsc.VectorSubcoreMesh(core_axis_name="c", subcore_axis_name="s")
    f = pl.kernel(
        _phase_router_kernel,
        mesh=mesh,
        compiler_params=pltpu.CompilerParams(needs_layout_passes=False),
        out_type=(
            jax.ShapeDtypeStruct((ROWS, NUM_EXPERTS), jnp.float32),
            jax.ShapeDtypeStruct((ROWS,), jnp.int32),
        ),
        scratch_types=[
            pltpu.VMEM((2, CH, COLS), jnp.int32),
            pltpu.VMEM((LANES, LANES), jnp.int32),
            pltpu.VMEM((ROWS_PER_W, NUM_EXPERTS), jnp.float32),
            pltpu.VMEM((ROWS_PER_W,), jnp.int32),
            pltpu.SemaphoreType.DMA,
            pltpu.SemaphoreType.DMA,
        ],
    )
    probs, idx = f(xi)
    return probs, idx
